# Initial kernel scaffold; baseline (speedup 1.0000x reference)
import jax
import jax.numpy as jnp
from jax.experimental import pallas as pl


def kernel(observation, mem_keys, mem_qvals, W1, b1, W2, b2, W3, b3):
    raise NotImplementedError("write your pallas kernel here")


# speedup vs baseline: 6.7347x; 6.7347x over previous
import functools

import jax
import jax.numpy as jnp
from jax import lax
from jax.experimental import pallas as pl
from jax.experimental.pallas import tpu as pltpu
from jax.experimental.pallas import tpu_sc as plsc

_BIG = 1e30
_CHUNK = 128
_GB = 32
_NCC = 512
_CCAP = 1024
_KNN = 32


def _k1_body(obs_ref, keys_ref, w1_ref, b1_ref, w2_ref, b2_ref, w3_ref, b3_ref,
             d2_ref, minis_ref, qnet_ref, *, cap, ck):
    j = pl.program_id(0)
    obs = obs_ref[...]
    keys = keys_ref[...]
    q_sq = jnp.sum(obs * obs, axis=1, keepdims=True)
    k_sq = jnp.sum(keys * keys, axis=1)[None, :]
    cross = lax.dot_general(obs, keys, (((1,), (1,)), ((), ())),
                            preferred_element_type=jnp.float32)
    d2 = (q_sq - 2.0 * cross) + k_sq
    col = j * ck + lax.broadcasted_iota(jnp.int32, d2.shape, 1)
    d2 = jnp.where(col >= cap, _BIG, d2)
    d2_ref[...] = d2
    nch = ck // _CHUNK
    minis_ref[...] = jnp.concatenate(
        [jnp.min(d2[:, c * _CHUNK:(c + 1) * _CHUNK], axis=1, keepdims=True)
         for c in range(nch)], axis=1)[None]
    h = jnp.maximum(lax.dot_general(obs, w1_ref[...], (((1,), (0,)), ((), ())),
                                    preferred_element_type=jnp.float32)
                    + b1_ref[0:1, :], 0.0)
    h = jnp.maximum(lax.dot_general(h, w2_ref[...], (((1,), (0,)), ((), ())),
                                    preferred_element_type=jnp.float32)
                    + b2_ref[0:1, :], 0.0)
    qnet_ref[...] = lax.dot_general(h, w3_ref[...], (((1,), (0,)), ((), ())),
                                    preferred_element_type=jnp.float32) + b3_ref[0:1, :]


def _dist_stage(obs, keys_p, w1, b1r, w2, b2r, w3p, b3r, *, cap, bm, ck,
                interpret=False):
    b, d = obs.shape
    cpad = keys_p.shape[0]
    hid = w1.shape[1]
    nj, ni = cpad // ck, b // bm
    nch = ck // _CHUNK
    nchunk = cpad // _CHUNK
    return pl.pallas_call(
        functools.partial(_k1_body, cap=cap, ck=ck),
        grid=(nj, ni),
        in_specs=[
            pl.BlockSpec((bm, d), lambda j, i: (i, 0)),
            pl.BlockSpec((ck, d), lambda j, i: (j, 0)),
            pl.BlockSpec((d, hid), lambda j, i: (0, 0)),
            pl.BlockSpec((8, hid), lambda j, i: (0, 0)),
            pl.BlockSpec((hid, hid), lambda j, i: (0, 0)),
            pl.BlockSpec((8, hid), lambda j, i: (0, 0)),
            pl.BlockSpec((hid, 128), lambda j, i: (0, 0)),
            pl.BlockSpec((8, 128), lambda j, i: (0, 0)),
        ],
        out_specs=[
            pl.BlockSpec((bm, ck), lambda j, i: (i, j)),
            pl.BlockSpec((1, bm, nch), lambda j, i: (j, i, 0)),
            pl.BlockSpec((bm, 128), lambda j, i: (i, 0)),
        ],
        out_shape=[
            jax.ShapeDtypeStruct((b, cpad), jnp.float32),
            jax.ShapeDtypeStruct((nj, b, nch), jnp.float32),
            jax.ShapeDtypeStruct((b, 128), jnp.float32),
        ],
        interpret=interpret,
    )(obs, keys_p, w1, b1r, w2, b2r, w3p, b3r)


def _knn_stage(d2c, minis, qvals_p, qnet, *, ac, interpret=False):
    b, nchunk = minis.shape
    nw = 32
    rpw = b // nw
    nvm = nchunk // 16
    mesh = plsc.VectorSubcoreMesh(core_axis_name="c", subcore_axis_name="s",
                                  num_cores=2, num_subcores=16)

    @functools.partial(
        pl.kernel,
        out_type=[jax.ShapeDtypeStruct((b, 16), jnp.float32),
                  jax.ShapeDtypeStruct((b,), jnp.int32)],
        mesh=mesh,
        scratch_types=[
            pltpu.VMEM((nchunk,), jnp.float32),
            pltpu.VMEM((_NCC,), jnp.int32),
            pltpu.VMEM((_NCC, _CHUNK), jnp.float32),
            pltpu.VMEM((_CCAP + 16,), jnp.float32),
            pltpu.VMEM((_CCAP + 16,), jnp.int32),
            pltpu.VMEM((_KNN,), jnp.float32),
            pltpu.VMEM((_KNN,), jnp.int32),
            pltpu.VMEM((_KNN,), jnp.float32),
            pltpu.VMEM((_KNN, 128), jnp.float32),
            pltpu.VMEM((rpw, 16), jnp.float32),
            pltpu.VMEM((rpw,), jnp.int32),
            pltpu.VMEM((128,), jnp.float32),
            pltpu.SemaphoreType.DMA,
        ],
        compiler_params=pltpu.CompilerParams(needs_layout_passes=False),
        interpret=interpret,
    )
    def k2(d2c_h, minis_h, qvals_h, qnet_h, q16_h, act_h,
           mrow, cidx, gbuf, cval, cgix, vbuf, ibuf, wbuf, qrows, qout, actb,
           qn128, sem):
        cci = lax.axis_index("c")
        ssi = lax.axis_index("s")
        wid = ssi * 2 + cci
        base = wid * rpw
        iota16 = lax.iota(jnp.int32, 16)
        lane0 = iota16 == 0

        def row_body(rl, carry):
            r = base + rl
            rn = r * nchunk
            pltpu.sync_copy(minis_h.at[r], mrow)

            def pa(i, c):
                m1, m2 = c
                v = mrow[pl.ds(i * 16, 16)]
                lo = jnp.minimum(m1, v)
                hi = jnp.maximum(m1, v)
                return lo, jnp.minimum(m2, hi)
            _, m2 = lax.fori_loop(0, nvm, pa,
                                  (jnp.full((16,), _BIG, jnp.float32),
                                   jnp.full((16,), _BIG, jnp.float32)))
            t0 = jnp.max(m2)

            padid = rn + (nchunk - 1)
            def pre(i, _):
                cidx[pl.ds(i * 16, 16)] = jnp.broadcast_to(padid, (16,))
                return 0
            lax.fori_loop(0, _NCC // 16, pre, 0)

            def pb(i, cnt):
                v = mrow[pl.ds(i * 16, 16)]
                m = v <= t0
                pc = jnp.sum(m.astype(jnp.int32))
                ids = (rn + i * 16) + iota16
                plsc.store_compressed(cidx.at[pl.ds(jnp.minimum(cnt, _NCC - 16), 16)],
                                      ids, mask=m)
                return jnp.minimum(cnt + pc, _NCC - 16)
            cnt = lax.fori_loop(0, nvm, pb, jnp.int32(0))

            nblk = (cnt + (_GB - 1)) >> 5
            def gf(bi, _):
                o = bi * _GB
                pltpu.async_copy(d2c_h.at[cidx.at[pl.ds(o, _GB)]],
                                 gbuf.at[pl.ds(o, _GB)], sem)
                return 0
            lax.fori_loop(0, nblk, gf, 0)
            def gd(bi, _):
                pltpu.make_async_copy(d2c_h.at[cidx.at[pl.ds(0, _GB)]],
                                      gbuf.at[pl.ds(0, _GB)], sem).wait()
                return 0
            lax.fori_loop(0, nblk, gd, 0)

            def pc_(jc, ccnt):
                jv = jnp.broadcast_to(jc, (16,))
                cidv = plsc.load_gather(cidx, [jv])
                ebase = (cidv - rn) * _CHUNK
                for u in range(_CHUNK // 16):
                    v = plsc.load_gather(gbuf, [jv, (u * 16) + iota16])
                    m = v <= t0
                    pcn = jnp.sum(m.astype(jnp.int32))
                    ei = ebase + ((u * 16) + iota16)
                    cc = jnp.minimum(ccnt, _CCAP - 16)
                    plsc.store_compressed(cval.at[pl.ds(cc, 16)], v, mask=m)
                    plsc.store_compressed(cgix.at[pl.ds(cc, 16)], ei, mask=m)
                    ccnt = jnp.minimum(ccnt + pcn, _CCAP - 16)
                return ccnt
            ccnt = lax.fori_loop(0, cnt, pc_, jnp.int32(0))
            cval[pl.ds(ccnt, 16)] = jnp.full((16,), _BIG, jnp.float32)
            nvc = (ccnt + 15) >> 4

            def ext(k, _):
                def sc1(i, mn):
                    return jnp.minimum(mn, cval[pl.ds(i * 16, 16)])
                mnv = lax.fori_loop(0, nvc, sc1, jnp.full((16,), _BIG, jnp.float32))
                mval = jnp.min(mnv)
                def sc2(i, pos):
                    v = cval[pl.ds(i * 16, 16)]
                    p = jnp.min(jnp.where(v == mval, (i * 16) + iota16,
                                          jnp.int32(2**30)))
                    return jnp.minimum(pos, p)
                pos = lax.fori_loop(0, nvc, sc2, jnp.int32(2**30))
                posv = jnp.broadcast_to(pos, (16,))
                gv = plsc.load_gather(cgix, [posv])
                kv = jnp.broadcast_to(k, (16,))
                plsc.store_scatter(vbuf, [kv], jnp.broadcast_to(mval, (16,)), mask=lane0)
                plsc.store_scatter(ibuf, [kv], gv, mask=lane0)
                plsc.store_scatter(cval, [posv],
                                   jnp.full((16,), _BIG, jnp.float32), mask=lane0)
                return 0
            lax.fori_loop(0, _KNN, ext, 0)

            v0 = vbuf[pl.ds(0, 16)]
            v1 = vbuf[pl.ds(16, 16)]
            w0 = 1.0 / (v0 + 1e-3)
            w1 = 1.0 / (v1 + 1e-3)
            s = jnp.sum(w0) + jnp.sum(w1)
            wbuf[pl.ds(0, 16)] = w0 / s
            wbuf[pl.ds(16, 16)] = w1 / s

            pltpu.async_copy(qvals_h.at[ibuf], qrows, sem).wait()
            qa = jnp.zeros((16,), jnp.float32)
            for jn in range(_KNN):
                wv = plsc.load_gather(wbuf, [jnp.full((16,), jn, jnp.int32)])
                qv = plsc.load_gather(qrows, [jnp.full((16,), jn, jnp.int32),
                                              iota16])
                qa = qa + wv * qv

            pltpu.sync_copy(qnet_h.at[r], qn128)
            q = 0.5 * (qa + qn128[pl.ds(0, 16)])
            plsc.store_scatter(qout, [jnp.broadcast_to(rl, (16,)), iota16], q)
            qm = jnp.where(iota16 < ac, q, -_BIG)
            mx = jnp.max(qm)
            av = jnp.min(jnp.where(qm == mx, iota16, jnp.int32(64)))
            plsc.store_scatter(actb, [jnp.broadcast_to(rl, (16,))],
                               jnp.broadcast_to(av, (16,)), mask=lane0)
            return carry

        lax.fori_loop(0, rpw, row_body, 0)
        pltpu.sync_copy(qout, q16_h.at[pl.ds(base, rpw)])
        pltpu.sync_copy(actb, act_h.at[pl.ds(base, rpw)])

    return k2(d2c, minis, qvals_p, qnet)


def kernel(observation, mem_keys, mem_qvals, W1, b1, W2, b2, W3, b3):
    b, d = observation.shape
    cap = mem_keys.shape[0]
    ac = mem_qvals.shape[1]
    hid = W1.shape[1]
    ck = 2048
    bm = 256
    cpad = ((cap + _CHUNK) + ck - 1) // ck * ck
    keys_p = jnp.pad(mem_keys, ((0, cpad - cap), (0, 0)))
    qvals_p = jnp.pad(mem_qvals, ((0, cpad - cap), (0, 128 - ac)))
    b1r = jnp.broadcast_to(b1[None, :], (8, hid))
    b2r = jnp.broadcast_to(b2[None, :], (8, hid))
    w3p = jnp.pad(W3, ((0, 0), (0, 128 - ac)))
    b3r = jnp.broadcast_to(jnp.pad(b3, (0, 128 - ac))[None, :], (8, 128))
    d2, minis3, qnet = _dist_stage(observation, keys_p, W1, b1r, W2, b2r, w3p,
                                   b3r, cap=cap, bm=bm, ck=ck)
    minis = minis3.transpose(1, 0, 2).reshape(b, cpad // _CHUNK)
    d2c = d2.reshape(b * (cpad // _CHUNK), _CHUNK)
    q16, act = _knn_stage(d2c, minis, qvals_p, qnet, ac=ac)
    return q16[:, :ac], act
